# baseline (device time: 135581 ns/iter reference)
import jax
import jax.numpy as jnp
from jax import lax
from jax.experimental import pallas as pl
from jax.experimental.pallas import tpu as pltpu

N_DEV = 32
LOG2 = 5
B, Sq, Skv, Hq, Dh = 2, 256, 256, 128, 64
H_SH = Hq // N_DEV
D_MODEL = 512


def kernel(x, Wq, K_ext, V_ext, Wo):
    idx = lax.axis_index("i")
    K_loc = lax.dynamic_slice_in_dim(K_ext, idx * H_SH, H_SH, axis=2)
    V_loc = lax.dynamic_slice_in_dim(V_ext, idx * H_SH, H_SH, axis=2)
    K_loc = K_loc.transpose(0, 2, 1, 3).reshape(B * H_SH, Skv, Dh)
    V_loc = V_loc.transpose(0, 2, 1, 3).reshape(B * H_SH, Skv, Dh)
    x2 = x.reshape(B * Sq, D_MODEL)

    def body(x_ref, wq_ref, k_ref, v_ref, wo_ref, out_ref,
             ctx_ref, recv_ref, send_sems, recv_sems):
        my = lax.axis_index("i")

        barrier = pltpu.get_barrier_semaphore()
        for r in range(LOG2):
            partner = jnp.bitwise_xor(my, 1 << r)
            pl.semaphore_signal(
                barrier, inc=1,
                device_id=(partner,), device_id_type=pl.DeviceIdType.MESH,
            )
        pl.semaphore_wait(barrier, LOG2)

        q = jnp.dot(
            x_ref[...].astype(jnp.bfloat16),
            wq_ref[...].astype(jnp.bfloat16),
            preferred_element_type=jnp.float32,
        )

        qi = lax.broadcasted_iota(jnp.int32, (Sq, Skv), 0)
        ki = lax.broadcasted_iota(jnp.int32, (Sq, Skv), 1)
        mask = (jnp.abs(qi - ki) <= 128) | (ki < 32) | (qi < 32)

        for b in range(B):
            for h in range(H_SH):
                bh = b * H_SH + h
                qbh = q[b * Sq:(b + 1) * Sq, h * Dh:(h + 1) * Dh]
                s = lax.dot_general(
                    qbh.astype(jnp.bfloat16),
                    k_ref[bh].astype(jnp.bfloat16),
                    (((1,), (1,)), ((), ())),
                    preferred_element_type=jnp.float32,
                ) * 0.125
                s = jnp.where(mask, s, -1e9)
                m = jnp.max(s, axis=1, keepdims=True)
                w = jnp.exp(s - m)
                w = w / jnp.sum(w, axis=1, keepdims=True)
                ctx_ref[b * Sq:(b + 1) * Sq, h * Dh:(h + 1) * Dh] = jnp.dot(
                    w.astype(jnp.bfloat16),
                    v_ref[bh].astype(jnp.bfloat16),
                    preferred_element_type=jnp.float32,
                )

        out_ref[...] = jnp.dot(
            ctx_ref[...].astype(jnp.bfloat16),
            wo_ref[...].astype(jnp.bfloat16),
            preferred_element_type=jnp.float32,
        )

        for r in range(LOG2):
            partner = jnp.bitwise_xor(my, 1 << r)
            rdma = pltpu.make_async_remote_copy(
                src_ref=out_ref,
                dst_ref=recv_ref.at[r],
                send_sem=send_sems.at[r],
                recv_sem=recv_sems.at[r],
                device_id=(partner,),
                device_id_type=pl.DeviceIdType.MESH,
            )
            rdma.start()
            rdma.wait()
            out_ref[...] += recv_ref[r]

    out = pl.pallas_call(
        body,
        out_shape=jax.ShapeDtypeStruct((B * Sq, D_MODEL), jnp.float32),
        in_specs=[pl.BlockSpec(memory_space=pltpu.VMEM)] * 5,
        out_specs=pl.BlockSpec(memory_space=pltpu.VMEM),
        scratch_shapes=[
            pltpu.VMEM((B * Sq, H_SH * Dh), jnp.float32),
            pltpu.VMEM((LOG2, B * Sq, D_MODEL), jnp.float32),
            pltpu.SemaphoreType.DMA((LOG2,)),
            pltpu.SemaphoreType.DMA((LOG2,)),
        ],
        compiler_params=pltpu.CompilerParams(collective_id=0),
    )(x2, Wq, K_loc, V_loc, Wo)
    return out.reshape(B, Sq, D_MODEL)


# device time: 99133 ns/iter; 1.3677x vs baseline; 1.3677x over previous
import jax
import jax.numpy as jnp
from jax import lax
from jax.experimental import pallas as pl
from jax.experimental.pallas import tpu as pltpu

N_DEV = 32
LOG2 = 5
B, Sq, Skv, Hq, Dh = 2, 256, 256, 128, 64
H_SH = Hq // N_DEV
D_MODEL = 512


def kernel(x, Wq, K_ext, V_ext, Wo):
    idx = lax.axis_index("i")
    K_loc = lax.dynamic_slice_in_dim(K_ext, idx * H_SH, H_SH, axis=2)
    V_loc = lax.dynamic_slice_in_dim(V_ext, idx * H_SH, H_SH, axis=2)
    x2 = x.reshape(B * Sq, D_MODEL)

    def body(x_ref, wq_ref, k_ref, v_ref, wo_ref, out_ref,
             ctx_ref, send_ref, recv_ref, send_sems, recv_sems):
        my = lax.axis_index("i")

        barrier = pltpu.get_barrier_semaphore()
        for r in range(LOG2):
            partner = jnp.bitwise_xor(my, 1 << r)
            pl.semaphore_signal(
                barrier, inc=1,
                device_id=(partner,), device_id_type=pl.DeviceIdType.MESH,
            )
        pl.semaphore_wait(barrier, LOG2)

        q = jnp.dot(
            x_ref[...].astype(jnp.bfloat16),
            wq_ref[...].astype(jnp.bfloat16),
            preferred_element_type=jnp.float32,
        )

        qi = lax.broadcasted_iota(jnp.int32, (Sq, Skv), 0)
        ki = lax.broadcasted_iota(jnp.int32, (Sq, Skv), 1)
        mask = (jnp.abs(qi - ki) <= 128) | (ki < 32) | (qi < 32)

        for b in range(B):
            for h in range(H_SH):
                bh = b * H_SH + h
                qbh = q[b * Sq:(b + 1) * Sq, h * Dh:(h + 1) * Dh]
                s = lax.dot_general(
                    qbh.astype(jnp.bfloat16),
                    k_ref[b, :, h, :].astype(jnp.bfloat16),
                    (((1,), (1,)), ((), ())),
                    preferred_element_type=jnp.float32,
                ) * 0.125
                s = jnp.where(mask, s, -1e9)
                m = jnp.max(s, axis=1, keepdims=True)
                w = jnp.exp(s - m)
                w = w / jnp.sum(w, axis=1, keepdims=True)
                ctx_ref[b * Sq:(b + 1) * Sq, h * Dh:(h + 1) * Dh] = jnp.dot(
                    w.astype(jnp.bfloat16),
                    v_ref[b, :, h, :].astype(jnp.bfloat16),
                    preferred_element_type=jnp.float32,
                )

        out_ref[...] = jnp.dot(
            ctx_ref[...].astype(jnp.bfloat16),
            wo_ref[...].astype(jnp.bfloat16),
            preferred_element_type=jnp.float32,
        )

        for r in range(LOG2):
            partner = jnp.bitwise_xor(my, 1 << r)
            send_ref[...] = out_ref[...].astype(jnp.bfloat16)
            rdma = pltpu.make_async_remote_copy(
                src_ref=send_ref,
                dst_ref=recv_ref.at[r],
                send_sem=send_sems.at[r],
                recv_sem=recv_sems.at[r],
                device_id=(partner,),
                device_id_type=pl.DeviceIdType.MESH,
            )
            rdma.start()
            rdma.wait()
            out_ref[...] += recv_ref[r].astype(jnp.float32)

    out = pl.pallas_call(
        body,
        out_shape=jax.ShapeDtypeStruct((B * Sq, D_MODEL), jnp.float32),
        in_specs=[pl.BlockSpec(memory_space=pltpu.VMEM)] * 5,
        out_specs=pl.BlockSpec(memory_space=pltpu.VMEM),
        scratch_shapes=[
            pltpu.VMEM((B * Sq, H_SH * Dh), jnp.float32),
            pltpu.VMEM((B * Sq, D_MODEL), jnp.bfloat16),
            pltpu.VMEM((LOG2, B * Sq, D_MODEL), jnp.bfloat16),
            pltpu.SemaphoreType.DMA((LOG2,)),
            pltpu.SemaphoreType.DMA((LOG2,)),
        ],
        compiler_params=pltpu.CompilerParams(collective_id=0),
    )(x2, Wq, K_loc, V_loc, Wo)
    return out.reshape(B, Sq, D_MODEL)
